# baseline (speedup 1.0000x reference)
import functools

import jax
import jax.numpy as jnp
from jax import lax
from jax.experimental import pallas as pl
from jax.experimental.pallas import tpu as pltpu
from jax.experimental.pallas import tpu_sc as plsc

T_TILE = 512
K = 2048
D = 256
NQ = 7
NTOK = 16384
NT = NTOK // T_TILE


def _stage_body(res_ref, cb_ref, a2_ref, b2_ref, idx_ref):
    r = res_ref[...]
    cb = cb_ref[...]
    S = lax.dot_general(r, cb, (((1,), (1,)), ((), ())),
                        preferred_element_type=jnp.float32)
    a2 = jnp.transpose(a2_ref[...], (1, 0))
    d2 = a2 + b2_ref[...] - 2.0 * S
    m2 = jnp.min(d2, axis=1, keepdims=True)
    s = jnp.sqrt(jnp.maximum(m2, 0.0))
    ulp = lax.bitcast_convert_type(
        lax.bitcast_convert_type(s, jnp.int32) + 1, jnp.float32) - s
    c = 4097.0 * s
    sh = c - (c - s)
    sl = s - sh
    p = s * s
    e = ((sh * sh - p) + 2.0 * (sh * sl)) + sl * sl
    t = e + s * ulp + 0.25 * (ulp * ulp)
    ks = lax.broadcasted_iota(jnp.int32, (T_TILE, K), 1)
    idx_ref[0, 0, :] = jnp.min(jnp.where((d2 - p) < t, ks, K), axis=1)


def _final_body(x_ref, res_ref, rows_ref, out_ref):
    out_ref[...] = x_ref[...] - (res_ref[...] - rows_ref[...])


@functools.cache
def _make_tc_calls():
    tok_spec = pl.BlockSpec((T_TILE, D), lambda i: (i, 0))
    cb_spec = pl.BlockSpec((K, D), lambda i: (0, 0))
    a2_spec = pl.BlockSpec((1, T_TILE), lambda i: (0, i))
    b2_spec = pl.BlockSpec((1, K), lambda i: (0, 0))

    stage_call = pl.pallas_call(
        _stage_body, grid=(NT,),
        in_specs=[tok_spec, cb_spec, a2_spec, b2_spec],
        out_specs=pl.BlockSpec((1, 1, T_TILE), lambda i: (i, 0, 0)),
        out_shape=jax.ShapeDtypeStruct((NT, 1, T_TILE), jnp.int32))

    final_call = pl.pallas_call(
        _final_body, grid=(NT,),
        in_specs=[tok_spec, tok_spec, tok_spec],
        out_specs=tok_spec,
        out_shape=jax.ShapeDtypeStruct((NTOK, D), jnp.float32))

    return stage_call, final_call


_NC = 2
_NS = 16
_NW = _NC * _NS
_TOK_W = NTOK // _NW
_CH = 128
_NCH = _TOK_W // _CH


@functools.cache
def _make_sc_gather():
    mesh = plsc.VectorSubcoreMesh(core_axis_name="c", subcore_axis_name="s")

    @functools.partial(
        pl.kernel, mesh=mesh,
        out_type=jax.ShapeDtypeStruct((NTOK, D), jnp.float32),
        scratch_types=[
            pltpu.VMEM((_CH,), jnp.int32),
            pltpu.VMEM((_CH, D), jnp.float32),
            pltpu.SemaphoreType.DMA,
        ],
    )
    def _sc_gather(cb_hbm, idx_hbm, out_hbm, idx_v, rows_v, sem):
        wid = lax.axis_index("s") * _NC + lax.axis_index("c")
        base = wid * _TOK_W
        for c in range(_NCH):
            off = base + c * _CH
            pltpu.sync_copy(idx_hbm.at[pl.ds(off, _CH)], idx_v)
            pltpu.async_copy(cb_hbm.at[idx_v], rows_v, sem).wait()
            pltpu.sync_copy(rows_v, out_hbm.at[pl.ds(off, _CH)])

    return _sc_gather


def kernel(x, codebooks):
    B, T, d = x.shape
    stage_call, final_call = _make_tc_calls()
    xf = x.reshape(B * T, d)
    indices = []
    res = xf
    rows = None
    for q in range(NQ):
        cb = codebooks[q]
        a2 = jnp.sum(res.reshape(B, T, d) * res.reshape(B, T, d),
                     axis=-1, keepdims=True)
        b2 = jnp.sum(cb * cb, axis=-1)
        idx3 = stage_call(res, cb, a2.reshape(1, NTOK), b2.reshape(1, K))
        idxf = idx3.reshape(NTOK)
        indices.append(idxf.reshape(B, T))
        new_rows = _make_sc_gather()(cb, idxf)
        if q < NQ - 1:
            res = res - new_rows
        rows = new_rows
    quant = final_call(xf, res, rows)
    return jnp.stack(indices, axis=0), quant.reshape(B, T, d)


# speedup vs baseline: 1.8374x; 1.8374x over previous
import functools

import jax
import jax.numpy as jnp
from jax import lax
from jax.experimental import pallas as pl
from jax.experimental.pallas import tpu as pltpu
from jax.experimental.pallas import tpu_sc as plsc

T_TILE = 1024
K = 2048
D = 256
NQ = 7
NTOK = 16384
NT = NTOK // T_TILE


def _stage_body(res_ref, cb_ref, a2_ref, b2_ref, idx_ref):
    r = res_ref[...]
    cb = cb_ref[...]
    S = lax.dot_general(r, cb, (((1,), (1,)), ((), ())),
                        preferred_element_type=jnp.float32)
    a2 = jnp.transpose(a2_ref[...], (1, 0))
    d2 = a2 + b2_ref[...] - 2.0 * S
    d = jnp.sqrt(jnp.maximum(d2, 0.0))
    m = jnp.min(d, axis=1, keepdims=True)
    ks = lax.broadcasted_iota(jnp.int32, (T_TILE, K), 1)
    idx_ref[0, 0, :] = jnp.min(jnp.where(d == m, ks, K), axis=1)


def _final_body(x_ref, res_ref, rows_ref, out_ref):
    out_ref[...] = x_ref[...] - (res_ref[...] - rows_ref[...])


@functools.cache
def _make_tc_calls():
    tok_spec = pl.BlockSpec((T_TILE, D), lambda i: (i, 0))
    cb_spec = pl.BlockSpec((K, D), lambda i: (0, 0))
    a2_spec = pl.BlockSpec((1, T_TILE), lambda i: (0, i))
    b2_spec = pl.BlockSpec((1, K), lambda i: (0, 0))

    stage_call = pl.pallas_call(
        _stage_body, grid=(NT,),
        in_specs=[tok_spec, cb_spec, a2_spec, b2_spec],
        out_specs=pl.BlockSpec((1, 1, T_TILE), lambda i: (i, 0, 0)),
        out_shape=jax.ShapeDtypeStruct((NT, 1, T_TILE), jnp.int32))

    final_call = pl.pallas_call(
        _final_body, grid=(NT,),
        in_specs=[tok_spec, tok_spec, tok_spec],
        out_specs=tok_spec,
        out_shape=jax.ShapeDtypeStruct((NTOK, D), jnp.float32))

    return stage_call, final_call


_NC = 2
_NS = 16
_NW = _NC * _NS
_TOK_W = NTOK // _NW
_CH = 128
_NCH = _TOK_W // _CH


@functools.cache
def _make_sc_gather():
    mesh = plsc.VectorSubcoreMesh(core_axis_name="c", subcore_axis_name="s")

    @functools.partial(
        pl.kernel, mesh=mesh,
        out_type=jax.ShapeDtypeStruct((NTOK, D), jnp.float32),
        scratch_types=[
            pltpu.VMEM((_CH,), jnp.int32),
            pltpu.VMEM((_CH, D), jnp.float32),
            pltpu.SemaphoreType.DMA,
        ],
    )
    def _sc_gather(cb_hbm, idx_hbm, out_hbm, idx_v, rows_v, sem):
        wid = lax.axis_index("s") * _NC + lax.axis_index("c")
        base = wid * _TOK_W
        for c in range(_NCH):
            off = base + c * _CH
            pltpu.sync_copy(idx_hbm.at[pl.ds(off, _CH)], idx_v)
            pltpu.async_copy(cb_hbm.at[idx_v], rows_v, sem).wait()
            pltpu.sync_copy(rows_v, out_hbm.at[pl.ds(off, _CH)])

    return _sc_gather


def kernel(x, codebooks):
    B, T, d = x.shape
    stage_call, final_call = _make_tc_calls()
    xf = x.reshape(B * T, d)
    indices = []
    res = xf
    rows = None
    for q in range(NQ):
        cb = codebooks[q]
        a2 = jnp.sum(res.reshape(B, T, d) * res.reshape(B, T, d),
                     axis=-1, keepdims=True)
        b2 = jnp.sum(cb * cb, axis=-1)
        idx3 = stage_call(res, cb, a2.reshape(1, NTOK), b2.reshape(1, K))
        idxf = idx3.reshape(NTOK)
        indices.append(idxf.reshape(B, T))
        new_rows = _make_sc_gather()(cb, idxf)
        if q < NQ - 1:
            res = res - new_rows
        rows = new_rows
    quant = final_call(xf, res, rows)
    return jnp.stack(indices, axis=0), quant.reshape(B, T, d)


# speedup vs baseline: 1.9149x; 1.0422x over previous
import functools

import jax
import jax.numpy as jnp
from jax import lax
from jax.experimental import pallas as pl
from jax.experimental.pallas import tpu as pltpu
from jax.experimental.pallas import tpu_sc as plsc

T_TILE = 2048
K = 2048
D = 256
NQ = 7
NTOK = 16384
NT = NTOK // T_TILE


def _stage_body(res_ref, cb_ref, a2_ref, b2_ref, idx_ref):
    r = res_ref[...]
    cb = cb_ref[...]
    S = lax.dot_general(r, cb, (((1,), (1,)), ((), ())),
                        preferred_element_type=jnp.float32)
    a2 = jnp.transpose(a2_ref[...], (1, 0))
    d2 = a2 + b2_ref[...] - 2.0 * S
    d = jnp.sqrt(jnp.maximum(d2, 0.0))
    m = jnp.min(d, axis=1, keepdims=True)
    ks = lax.broadcasted_iota(jnp.int32, (T_TILE, K), 1)
    idx_ref[0, 0, :] = jnp.min(jnp.where(d == m, ks, K), axis=1)


def _final_body(x_ref, res_ref, rows_ref, out_ref):
    out_ref[...] = x_ref[...] - (res_ref[...] - rows_ref[...])


@functools.cache
def _make_tc_calls():
    tok_spec = pl.BlockSpec((T_TILE, D), lambda i: (i, 0))
    cb_spec = pl.BlockSpec((K, D), lambda i: (0, 0))
    a2_spec = pl.BlockSpec((1, T_TILE), lambda i: (0, i))
    b2_spec = pl.BlockSpec((1, K), lambda i: (0, 0))

    stage_call = pl.pallas_call(
        _stage_body, grid=(NT,),
        in_specs=[tok_spec, cb_spec, a2_spec, b2_spec],
        out_specs=pl.BlockSpec((1, 1, T_TILE), lambda i: (i, 0, 0)),
        out_shape=jax.ShapeDtypeStruct((NT, 1, T_TILE), jnp.int32))

    final_call = pl.pallas_call(
        _final_body, grid=(NT,),
        in_specs=[tok_spec, tok_spec, tok_spec],
        out_specs=tok_spec,
        out_shape=jax.ShapeDtypeStruct((NTOK, D), jnp.float32))

    return stage_call, final_call


_NC = 2
_NS = 16
_NW = _NC * _NS
_TOK_W = NTOK // _NW
_CH = 128
_NCH = _TOK_W // _CH


@functools.cache
def _make_sc_gather():
    mesh = plsc.VectorSubcoreMesh(core_axis_name="c", subcore_axis_name="s")

    @functools.partial(
        pl.kernel, mesh=mesh,
        out_type=jax.ShapeDtypeStruct((NTOK, D), jnp.float32),
        scratch_types=[
            pltpu.VMEM((_CH,), jnp.int32),
            pltpu.VMEM((_CH, D), jnp.float32),
            pltpu.SemaphoreType.DMA,
        ],
    )
    def _sc_gather(cb_hbm, idx_hbm, out_hbm, idx_v, rows_v, sem):
        wid = lax.axis_index("s") * _NC + lax.axis_index("c")
        base = wid * _TOK_W
        for c in range(_NCH):
            off = base + c * _CH
            pltpu.sync_copy(idx_hbm.at[pl.ds(off, _CH)], idx_v)
            pltpu.async_copy(cb_hbm.at[idx_v], rows_v, sem).wait()
            pltpu.sync_copy(rows_v, out_hbm.at[pl.ds(off, _CH)])

    return _sc_gather


def kernel(x, codebooks):
    B, T, d = x.shape
    stage_call, final_call = _make_tc_calls()
    xf = x.reshape(B * T, d)
    indices = []
    res = xf
    rows = None
    for q in range(NQ):
        cb = codebooks[q]
        a2 = jnp.sum(res.reshape(B, T, d) * res.reshape(B, T, d),
                     axis=-1, keepdims=True)
        b2 = jnp.sum(cb * cb, axis=-1)
        idx3 = stage_call(res, cb, a2.reshape(1, NTOK), b2.reshape(1, K))
        idxf = idx3.reshape(NTOK)
        indices.append(idxf.reshape(B, T))
        new_rows = _make_sc_gather()(cb, idxf)
        if q < NQ - 1:
            res = res - new_rows
        rows = new_rows
    quant = final_call(xf, res, rows)
    return jnp.stack(indices, axis=0), quant.reshape(B, T, d)
